# baseline (speedup 1.0000x reference)
import functools

import jax
import jax.numpy as jnp
from jax import lax
from jax.experimental import pallas as pl
from jax.experimental.pallas import tpu as pltpu
from jax.experimental.pallas import tpu_sc as plsc

BATCH = 16384
DIM = 64
NENT = 1000000
NREL = 1000

_GATHER_DNUMS = lax.GatherDimensionNumbers(
    offset_dims=(), collapsed_slice_dims=(0,), start_index_map=(0,)
)


def _vperm(v, p):
    return lax.gather(
        v,
        p.reshape(16, 1),
        _GATHER_DNUMS,
        (1,),
        mode=lax.GatherScatterMode.PROMISE_IN_BOUNDS,
    )


def _make_perms():
    lanes = lax.iota(jnp.int32, 16)
    fold = {k: (lanes + k // 2) & 15 for k in (16, 8, 4, 2)}
    pack = {k: (lanes - k // 2) & 15 for k in (16, 8, 4, 2)}
    odd = {
        k: ((lanes >> {8: 3, 4: 2, 2: 1, 1: 0}[k // 2]) & 1).astype(jnp.float32)
        for k in (16, 8, 4, 2)
    }
    bitrev = (
        ((lanes & 1) << 3) | ((lanes & 2) << 1) | ((lanes & 4) >> 1) | ((lanes & 8) >> 3)
    )
    return fold, pack, odd, bitrev


def _rowsum16(vecs, perms):
    fold, pack, odd, bitrev = perms

    def merge(a, b, k):
        a2 = a + _vperm(a, fold[k])
        b2 = b + _vperm(b, fold[k])
        pb = _vperm(b2, pack[k])
        return a2 + (pb - a2) * odd[k]

    k = 16
    while len(vecs) > 1:
        vecs = [merge(vecs[2 * i], vecs[2 * i + 1], k) for i in range(len(vecs) // 2)]
        k //= 2
    return _vperm(vecs[0], bitrev)


def kernel(e, l, t, Ee, El):
    info = plsc.get_sparse_core_info()
    nc, ns, nl = info.num_cores, info.num_subcores, info.num_lanes
    nw = nc * ns
    bpw = BATCH // nw
    chunk = 256
    nchunks = bpw // chunk

    ee2 = Ee.reshape(NENT // 2, 2 * DIM)
    el2 = El.reshape(NREL // 2, 2 * DIM)

    mesh = plsc.VectorSubcoreMesh(core_axis_name="c", subcore_axis_name="s")

    @functools.partial(
        pl.kernel,
        mesh=mesh,
        out_type=jax.ShapeDtypeStruct((BATCH,), jnp.float32),
        scratch_types=[
            pltpu.VMEM((bpw,), jnp.int32),
            pltpu.VMEM((bpw,), jnp.int32),
            pltpu.VMEM((bpw,), jnp.int32),
            pltpu.VMEM((bpw,), jnp.int32),
            pltpu.VMEM((bpw,), jnp.int32),
            pltpu.VMEM((bpw,), jnp.int32),
            pltpu.VMEM((chunk, 2 * DIM), jnp.float32),
            pltpu.VMEM((chunk, 2 * DIM), jnp.float32),
            pltpu.VMEM((chunk, 2 * DIM), jnp.float32),
            pltpu.VMEM((bpw,), jnp.float32),
            pltpu.SemaphoreType.DMA,
            pltpu.SemaphoreType.DMA,
            pltpu.SemaphoreType.DMA,
        ],
    )
    def trans_e(ee_hbm, el_hbm, e_hbm, l_hbm, t_hbm, out_hbm,
                ei_v, li_v, ti_v, ep_v, lp_v, tp_v, er_v, lr_v, tr_v, out_v,
                sem_e, sem_l, sem_t):
        wid = lax.axis_index("s") * nc + lax.axis_index("c")
        base = wid * bpw
        pltpu.sync_copy(e_hbm.at[pl.ds(base, bpw)], ei_v)
        pltpu.sync_copy(l_hbm.at[pl.ds(base, bpw)], li_v)
        pltpu.sync_copy(t_hbm.at[pl.ds(base, bpw)], ti_v)

        def pair_idx(i, carry):
            ds = pl.ds(i * nl, nl)
            ep_v[ds] = lax.shift_right_logical(ei_v[ds], 1)
            lp_v[ds] = lax.shift_right_logical(li_v[ds], 1)
            tp_v[ds] = lax.shift_right_logical(ti_v[ds], 1)
            return carry

        lax.fori_loop(0, bpw // nl, pair_idx, 0)

        def do_chunk(k, carry):
            koff = k * chunk
            ce = pltpu.async_copy(ee_hbm.at[ep_v.at[pl.ds(koff, chunk)]], er_v, sem_e)
            cl = pltpu.async_copy(el_hbm.at[lp_v.at[pl.ds(koff, chunk)]], lr_v, sem_l)
            ct = pltpu.async_copy(ee_hbm.at[tp_v.at[pl.ds(koff, chunk)]], tr_v, sem_t)
            ce.wait()
            cl.wait()
            ct.wait()

            def group(g, carry2):
                perms = _make_perms()
                g16 = g * nl
                pe16 = (ei_v[pl.ds(koff + g16, nl)] & 1).astype(jnp.float32)
                pl16 = (li_v[pl.ds(koff + g16, nl)] & 1).astype(jnp.float32)
                pt16 = (ti_v[pl.ds(koff + g16, nl)] & 1).astype(jnp.float32)
                rows = []
                for r in range(nl):
                    row = g16 + r
                    splat_r = jnp.full((nl,), r, jnp.int32)
                    pef = _vperm(pe16, splat_r)
                    plf = _vperm(pl16, splat_r)
                    ptf = _vperm(pt16, splat_r)
                    acc = None
                    for c in range(DIM // nl):
                        lo = pl.ds(c * nl, nl)
                        hi = pl.ds(DIM + c * nl, nl)
                        elo = er_v[row, lo]
                        llo = lr_v[row, lo]
                        tlo = tr_v[row, lo]
                        ev = elo + (er_v[row, hi] - elo) * pef
                        lv = llo + (lr_v[row, hi] - llo) * plf
                        tv = tlo + (tr_v[row, hi] - tlo) * ptf
                        d = jnp.abs(ev + lv - tv)
                        acc = d if acc is None else acc + d
                    rows.append(acc)
                out_v[pl.ds(koff + g16, nl)] = _rowsum16(rows, perms)
                return carry2

            lax.fori_loop(0, chunk // nl, group, 0)
            return carry

        lax.fori_loop(0, nchunks, do_chunk, 0)
        pltpu.sync_copy(out_v, out_hbm.at[pl.ds(base, bpw)])

    return trans_e(ee2, el2, e, l, t)


# speedup vs baseline: 1.0122x; 1.0122x over previous
import functools

import jax
import jax.numpy as jnp
from jax import lax
from jax.experimental import pallas as pl
from jax.experimental.pallas import tpu as pltpu
from jax.experimental.pallas import tpu_sc as plsc

BATCH = 16384
DIM = 64

_GATHER_DNUMS = lax.GatherDimensionNumbers(
    offset_dims=(), collapsed_slice_dims=(0,), start_index_map=(0,)
)


def _vperm(v, p):
    return lax.gather(
        v,
        p.reshape(16, 1),
        _GATHER_DNUMS,
        (1,),
        mode=lax.GatherScatterMode.PROMISE_IN_BOUNDS,
    )


def _make_perms():
    lanes = lax.iota(jnp.int32, 16)
    fold = {k: (lanes + k // 2) & 15 for k in (16, 8, 4, 2)}
    pack = {k: (lanes - k // 2) & 15 for k in (16, 8, 4, 2)}
    odd = {
        k: ((lanes >> {8: 3, 4: 2, 2: 1, 1: 0}[k // 2]) & 1).astype(jnp.float32)
        for k in (16, 8, 4, 2)
    }
    bitrev = (
        ((lanes & 1) << 3) | ((lanes & 2) << 1) | ((lanes & 4) >> 1) | ((lanes & 8) >> 3)
    )
    return fold, pack, odd, bitrev


def _rowsum16(vecs, perms):
    fold, pack, odd, bitrev = perms

    def merge(a, b, k):
        a2 = a + _vperm(a, fold[k])
        b2 = b + _vperm(b, fold[k])
        pb = _vperm(b2, pack[k])
        return a2 + (pb - a2) * odd[k]

    k = 16
    while len(vecs) > 1:
        vecs = [merge(vecs[2 * i], vecs[2 * i + 1], k) for i in range(len(vecs) // 2)]
        k //= 2
    return _vperm(vecs[0], bitrev)


def kernel(e, l, t, Ee, El):
    info = plsc.get_sparse_core_info()
    nc, ns, nl = info.num_cores, info.num_subcores, info.num_lanes
    nw = nc * ns
    bpw = BATCH // nw

    mesh = plsc.VectorSubcoreMesh(core_axis_name="c", subcore_axis_name="s")

    @functools.partial(
        pl.kernel,
        mesh=mesh,
        out_type=jax.ShapeDtypeStruct((BATCH,), jnp.float32),
        compiler_params=pltpu.CompilerParams(use_tc_tiling_on_sc=False),
        scratch_types=[
            pltpu.VMEM((bpw,), jnp.int32),
            pltpu.VMEM((bpw,), jnp.int32),
            pltpu.VMEM((bpw,), jnp.int32),
            pltpu.VMEM((bpw, DIM), jnp.float32),
            pltpu.VMEM((bpw, DIM), jnp.float32),
            pltpu.VMEM((bpw, DIM), jnp.float32),
            pltpu.VMEM((bpw,), jnp.float32),
            pltpu.SemaphoreType.DMA,
            pltpu.SemaphoreType.DMA,
            pltpu.SemaphoreType.DMA,
        ],
    )
    def trans_e(ee_hbm, el_hbm, e_hbm, l_hbm, t_hbm, out_hbm,
                ei_v, li_v, ti_v, er_v, lr_v, tr_v, out_v,
                sem_e, sem_l, sem_t):
        wid = lax.axis_index("s") * nc + lax.axis_index("c")
        base = wid * bpw
        pltpu.sync_copy(e_hbm.at[pl.ds(base, bpw)], ei_v)
        pltpu.sync_copy(l_hbm.at[pl.ds(base, bpw)], li_v)
        pltpu.sync_copy(t_hbm.at[pl.ds(base, bpw)], ti_v)
        ce = pltpu.async_copy(ee_hbm.at[ei_v], er_v, sem_e)
        cl = pltpu.async_copy(el_hbm.at[li_v], lr_v, sem_l)
        ct = pltpu.async_copy(ee_hbm.at[ti_v], tr_v, sem_t)
        ce.wait()
        cl.wait()
        ct.wait()

        def group(g, carry):
            perms = _make_perms()
            g16 = g * nl
            rows = []
            for r in range(nl):
                row = g16 + r
                acc = None
                for c in range(DIM // nl):
                    ds = pl.ds(c * nl, nl)
                    d = jnp.abs(er_v[row, ds] + lr_v[row, ds] - tr_v[row, ds])
                    acc = d if acc is None else acc + d
                rows.append(acc)
            out_v[pl.ds(g16, nl)] = _rowsum16(rows, perms)
            return carry

        lax.fori_loop(0, bpw // nl, group, 0)
        pltpu.sync_copy(out_v, out_hbm.at[pl.ds(base, bpw)])

    return trans_e(Ee, El, e, l, t)


# speedup vs baseline: 1.7059x; 1.6853x over previous
import functools

import jax
import jax.numpy as jnp
from jax import lax
from jax.experimental import pallas as pl
from jax.experimental.pallas import tpu as pltpu
from jax.experimental.pallas import tpu_sc as plsc

BATCH = 16384
DIM = 64

_GATHER_DNUMS = lax.GatherDimensionNumbers(
    offset_dims=(), collapsed_slice_dims=(0,), start_index_map=(0,)
)


def _vperm(v, p):
    return lax.gather(
        v,
        p.reshape(16, 1),
        _GATHER_DNUMS,
        (1,),
        mode=lax.GatherScatterMode.PROMISE_IN_BOUNDS,
    )


def _lane(v, r):
    splat = jnp.full((16,), r, jnp.int32)
    return lax.reduce_max(_vperm(v, splat), axes=(0,))


def _make_perms():
    lanes = lax.iota(jnp.int32, 16)
    fold = {k: (lanes + k // 2) & 15 for k in (16, 8, 4, 2)}
    pack = {k: (lanes - k // 2) & 15 for k in (16, 8, 4, 2)}
    odd = {
        k: ((lanes >> {8: 3, 4: 2, 2: 1, 1: 0}[k // 2]) & 1).astype(jnp.float32)
        for k in (16, 8, 4, 2)
    }
    bitrev = (
        ((lanes & 1) << 3) | ((lanes & 2) << 1) | ((lanes & 4) >> 1) | ((lanes & 8) >> 3)
    )
    return fold, pack, odd, bitrev


def _rowsum16(vecs, perms):
    fold, pack, odd, bitrev = perms

    def merge(a, b, k):
        a2 = a + _vperm(a, fold[k])
        b2 = b + _vperm(b, fold[k])
        pb = _vperm(b2, pack[k])
        return a2 + (pb - a2) * odd[k]

    k = 16
    while len(vecs) > 1:
        vecs = [merge(vecs[2 * i], vecs[2 * i + 1], k) for i in range(len(vecs) // 2)]
        k //= 2
    return _vperm(vecs[0], bitrev)


def kernel(e, l, t, Ee, El):
    info = plsc.get_sparse_core_info()
    nc, ns, nl = info.num_cores, info.num_subcores, info.num_lanes
    nw = nc * ns
    bpw = BATCH // nw
    chunk = 256
    nchunks = bpw // chunk

    mesh = plsc.VectorSubcoreMesh(core_axis_name="c", subcore_axis_name="s")

    @functools.partial(
        pl.kernel,
        mesh=mesh,
        out_type=jax.ShapeDtypeStruct((BATCH,), jnp.float32),
        compiler_params=pltpu.CompilerParams(needs_layout_passes=False),
        scratch_types=[
            pltpu.VMEM((bpw,), jnp.int32),
            pltpu.VMEM((bpw,), jnp.int32),
            pltpu.VMEM((bpw,), jnp.int32),
            pltpu.VMEM((chunk, DIM), jnp.float32),
            pltpu.VMEM((chunk, DIM), jnp.float32),
            pltpu.VMEM((chunk, DIM), jnp.float32),
            pltpu.VMEM((bpw,), jnp.float32),
            pltpu.SemaphoreType.DMA,
            pltpu.SemaphoreType.DMA,
            pltpu.SemaphoreType.DMA,
        ],
    )
    def trans_e(ee_hbm, el_hbm, e_hbm, l_hbm, t_hbm, out_hbm,
                ei_v, li_v, ti_v, er_v, lr_v, tr_v, out_v,
                sem_e, sem_l, sem_t):
        wid = lax.axis_index("s") * nc + lax.axis_index("c")
        base = wid * bpw
        pltpu.sync_copy(e_hbm.at[pl.ds(base, bpw)], ei_v)
        pltpu.sync_copy(l_hbm.at[pl.ds(base, bpw)], li_v)
        pltpu.sync_copy(t_hbm.at[pl.ds(base, bpw)], ti_v)

        def do_chunk(ck, carry):
            koff = ck * chunk

            def fire(g, carry2):
                g16 = g * nl
                iv_e = ei_v[pl.ds(koff + g16, nl)]
                iv_l = li_v[pl.ds(koff + g16, nl)]
                iv_t = ti_v[pl.ds(koff + g16, nl)]
                for r in range(nl):
                    row = g16 + r
                    pltpu.async_copy(ee_hbm.at[_lane(iv_e, r)], er_v.at[row], sem_e)
                    pltpu.async_copy(el_hbm.at[_lane(iv_l, r)], lr_v.at[row], sem_l)
                    pltpu.async_copy(ee_hbm.at[_lane(iv_t, r)], tr_v.at[row], sem_t)
                return carry2

            lax.fori_loop(0, chunk // nl, fire, 0)

            pltpu.make_async_copy(ee_hbm.at[pl.ds(0, chunk)], er_v, sem_e).wait()
            pltpu.make_async_copy(el_hbm.at[pl.ds(0, chunk)], lr_v, sem_l).wait()
            pltpu.make_async_copy(ee_hbm.at[pl.ds(0, chunk)], tr_v, sem_t).wait()

            def group(g, carry2):
                perms = _make_perms()
                g16 = g * nl
                rows = []
                for r in range(nl):
                    row = g16 + r
                    acc = None
                    for c in range(DIM // nl):
                        ds = pl.ds(c * nl, nl)
                        d = jnp.abs(er_v[row, ds] + lr_v[row, ds] - tr_v[row, ds])
                        acc = d if acc is None else acc + d
                    rows.append(acc)
                out_v[pl.ds(koff + g16, nl)] = _rowsum16(rows, perms)
                return carry2

            lax.fori_loop(0, chunk // nl, group, 0)
            return carry

        lax.fori_loop(0, nchunks, do_chunk, 0)
        pltpu.sync_copy(out_v, out_hbm.at[pl.ds(base, bpw)])

    return trans_e(Ee, El, e, l, t)
